# Initial kernel scaffold; baseline (speedup 1.0000x reference)
import jax
import jax.numpy as jnp
from jax.experimental import pallas as pl


def kernel(PPI_x, PPI_edge_index, PPI_batch, edge_attr, W_in, b_in, W1, b1, W2, b2, W3, b3, W_out, b_out):
    raise NotImplementedError("write your pallas kernel here")


# speedup vs baseline: 6.7683x; 6.7683x over previous
import functools

import jax
import jax.numpy as jnp
from jax import lax
from jax.experimental import pallas as pl
from jax.experimental.pallas import tpu as pltpu
from jax.experimental.pallas import tpu_sc as plsc

CH = 128
DEGW = 128
NSUB = 16
NCORE = 2


def _round_up(x, m):
    return (x + m - 1) // m * m


def _deg_body(n_acc, chunks_per_tile, ep_ref, h0_ref, h1_ref,
              eidx, ones_v, zbuf, acc):
    c = lax.axis_index("c")
    s = lax.axis_index("s")
    w = s * NCORE + c

    def _memset(i, _):
        for j in range(DEGW // 16):
            ones_v[i, pl.ds(j * 16, 16)] = jnp.ones((16,), jnp.float32)
            zbuf[i, pl.ds(j * 16, 16)] = jnp.zeros((16,), jnp.float32)
        return 0
    lax.fori_loop(0, CH, _memset, 0)

    zc = n_acc // NSUB // CH
    for b in range(zc):
        pltpu.sync_copy(zbuf, acc.at[pl.ds(s * (n_acc // NSUB) + b * CH, CH)])
    plsc.subcore_barrier()

    def _body(i, _):
        ch = w * chunks_per_tile + i
        pltpu.sync_copy(ep_ref.at[ch], eidx)
        pltpu.sync_copy(ones_v, acc.at[eidx.at[1]], add=True)
        return 0
    lax.fori_loop(0, chunks_per_tile, _body, 0)
    plsc.subcore_barrier()

    rows = n_acc // NSUB
    off = s * rows

    @pl.when(c == 0)
    def _():
        pltpu.sync_copy(acc.at[pl.ds(off, rows)], h0_ref.at[pl.ds(off, rows)])

    @pl.when(c == 1)
    def _():
        pltpu.sync_copy(acc.at[pl.ds(off, rows)], h1_ref.at[pl.ds(off, rows)])


def _conv_body(n_acc, dh, chunks_per_tile, y0_ref, y1_ref, ep_ref,
               o0_ref, o1_ref, eidx, rows_v, acc, sem):
    c = lax.axis_index("c")
    s = lax.axis_index("s")

    def _memset(i, _):
        for j in range(dh // 16):
            rows_v[i, pl.ds(j * 16, 16)] = jnp.zeros((16,), jnp.float32)
        return 0
    lax.fori_loop(0, CH, _memset, 0)

    zc = n_acc // NSUB // CH
    for b in range(zc):
        pltpu.sync_copy(rows_v, acc.at[pl.ds(s * (n_acc // NSUB) + b * CH, CH)])
    plsc.subcore_barrier()

    def _body(i, _):
        ch = s * chunks_per_tile + i
        pltpu.sync_copy(ep_ref.at[ch], eidx)

        @pl.when(c == 0)
        def _():
            pltpu.async_copy(y0_ref.at[eidx.at[0]], rows_v, sem).wait()

        @pl.when(c == 1)
        def _():
            pltpu.async_copy(y1_ref.at[eidx.at[0]], rows_v, sem).wait()

        pltpu.sync_copy(rows_v, acc.at[eidx.at[1]], add=True)
        return 0
    lax.fori_loop(0, chunks_per_tile, _body, 0)
    plsc.subcore_barrier()

    rows = n_acc // NSUB
    off = s * rows

    @pl.when(c == 0)
    def _():
        pltpu.sync_copy(acc.at[pl.ds(off, rows)], o0_ref.at[pl.ds(off, rows)])

    @pl.when(c == 1)
    def _():
        pltpu.sync_copy(acc.at[pl.ds(off, rows)], o1_ref.at[pl.ds(off, rows)])


def _mm1_body(dh, x_ref, w_ref, b_ref, w1_ref, h0_ref, h1_ref,
              z_ref, y0_ref, y1_ref, dinv_ref, acc_ref):
    k = pl.program_id(1)

    @pl.when(k == 0)
    def _():
        acc_ref[...] = jnp.zeros_like(acc_ref)

    acc_ref[...] += jnp.dot(x_ref[...], w_ref[...],
                            preferred_element_type=jnp.float32)

    @pl.when(k == pl.num_programs(1) - 1)
    def _():
        h = jnp.maximum(acc_ref[...] + b_ref[...], 0.0)
        deg = 2.0 + h0_ref[:, 0:1] + h1_ref[:, 0:1]
        dinv = lax.rsqrt(deg)
        z = jnp.dot(h, w1_ref[...], preferred_element_type=jnp.float32)
        z_ref[...] = z
        y = z * dinv
        y0_ref[...] = y[:, :dh]
        y1_ref[...] = y[:, dh:]
        dinv_ref[...] = dinv


def _mid_body(dh, s0_ref, s1_ref, z_ref, dinv_ref, b_ref, w_ref,
              zo_ref, y0_ref, y1_ref):
    dinv = dinv_ref[...]
    sc = jnp.concatenate([s0_ref[...], s1_ref[...]], axis=1)
    x = dinv * sc + (2.0 * dinv * dinv) * z_ref[...] + b_ref[...]
    z = jnp.dot(x, w_ref[...], preferred_element_type=jnp.float32)
    zo_ref[...] = z
    y = z * dinv
    y0_ref[...] = y[:, :dh]
    y1_ref[...] = y[:, dh:]


def _fin_body(s0_ref, s1_ref, z_ref, dinv_ref, b_ref, wo_ref, bo_ref, o_ref):
    dinv = dinv_ref[...]
    sc = jnp.concatenate([s0_ref[...], s1_ref[...]], axis=1)
    x = dinv * sc + (2.0 * dinv * dinv) * z_ref[...] + b_ref[...]
    o_ref[...] = jax.nn.sigmoid(
        jnp.dot(x, wo_ref[...], preferred_element_type=jnp.float32) + bo_ref[...])


def kernel(PPI_x, PPI_edge_index, PPI_batch, edge_attr, W_in, b_in,
           W1, b1, W2, b2, W3, b3, W_out, b_out):
    del PPI_batch, edge_attr
    n, din = PPI_x.shape
    d = W1.shape[0]
    e = PPI_edge_index.shape[1]

    dp = _round_up(d, 256)
    dh = dp // 2
    n_acc = _round_up(n + CH, NSUB * CH)
    e_pad = _round_up(e, NCORE * NSUB * CH)

    pad = e_pad - e
    src = PPI_edge_index[0]
    dst = PPI_edge_index[1]
    src_p = jnp.concatenate([src, jnp.zeros((pad,), jnp.int32)])
    dst_p = jnp.concatenate([dst, jnp.full((pad,), n, jnp.int32)])
    nch = e_pad // CH
    epack = jnp.stack([src_p, dst_p]).reshape(2, nch, CH).transpose(1, 0, 2)

    W_in_p = jnp.pad(W_in, ((0, 0), (0, dp - d)))
    b_in_p = jnp.pad(b_in, (0, dp - d)).reshape(1, dp)
    Wp = [jnp.pad(W, ((0, dp - d), (0, dp - d))) for W in (W1, W2, W3)]
    bp = [jnp.pad(b, (0, dp - d)).reshape(1, dp) for b in (b1, b2, b3)]
    W_out_p = jnp.pad(W_out, ((0, dp - d), (0, 0)))
    b_out_p = b_out.reshape(1, 1)

    f32 = jnp.float32
    mesh = plsc.VectorSubcoreMesh(core_axis_name="c", subcore_axis_name="s")

    deg_chunks = e_pad // (NCORE * NSUB * CH)
    deg_call = pl.kernel(
        functools.partial(_deg_body, n_acc, deg_chunks),
        out_type=[jax.ShapeDtypeStruct((n_acc, DEGW), f32)] * 2,
        mesh=mesh,
        scratch_types=[
            pltpu.VMEM((2, CH), jnp.int32),
            pltpu.VMEM((CH, DEGW), f32),
            pltpu.VMEM((CH, DEGW), f32),
            pltpu.VMEM_SHARED((n_acc, DEGW), f32),
        ],
    )
    h0, h1 = deg_call(epack)

    conv_chunks = e_pad // (NSUB * CH)
    conv_call = pl.kernel(
        functools.partial(_conv_body, n_acc, dh, conv_chunks),
        out_type=[jax.ShapeDtypeStruct((n_acc, dh), f32)] * 2,
        mesh=mesh,
        scratch_types=[
            pltpu.VMEM((2, CH), jnp.int32),
            pltpu.VMEM((CH, dh), f32),
            pltpu.VMEM_SHARED((n_acc, dh), f32),
            pltpu.SemaphoreType.DMA,
        ],
    )

    bm = 1000
    bk = 1024
    gm, gk = n // bm, din // bk
    z1, y0, y1, dinv = pl.pallas_call(
        functools.partial(_mm1_body, dh),
        grid=(gm, gk),
        in_specs=[
            pl.BlockSpec((bm, bk), lambda m, k: (m, k)),
            pl.BlockSpec((bk, dp), lambda m, k: (k, 0)),
            pl.BlockSpec((1, dp), lambda m, k: (0, 0)),
            pl.BlockSpec((dp, dp), lambda m, k: (0, 0)),
            pl.BlockSpec((bm, DEGW), lambda m, k: (m, 0)),
            pl.BlockSpec((bm, DEGW), lambda m, k: (m, 0)),
        ],
        out_specs=[
            pl.BlockSpec((bm, dp), lambda m, k: (m, 0)),
            pl.BlockSpec((bm, dh), lambda m, k: (m, 0)),
            pl.BlockSpec((bm, dh), lambda m, k: (m, 0)),
            pl.BlockSpec((bm, 1), lambda m, k: (m, 0)),
        ],
        out_shape=[
            jax.ShapeDtypeStruct((n, dp), f32),
            jax.ShapeDtypeStruct((n, dh), f32),
            jax.ShapeDtypeStruct((n, dh), f32),
            jax.ShapeDtypeStruct((n, 1), f32),
        ],
        scratch_shapes=[pltpu.VMEM((bm, dp), f32)],
    )(PPI_x, W_in_p, b_in_p, Wp[0], h0, h1)

    mid_call = pl.pallas_call(
        functools.partial(_mid_body, dh),
        grid=(gm,),
        in_specs=[
            pl.BlockSpec((bm, dh), lambda m: (m, 0)),
            pl.BlockSpec((bm, dh), lambda m: (m, 0)),
            pl.BlockSpec((bm, dp), lambda m: (m, 0)),
            pl.BlockSpec((bm, 1), lambda m: (m, 0)),
            pl.BlockSpec((1, dp), lambda m: (0, 0)),
            pl.BlockSpec((dp, dp), lambda m: (0, 0)),
        ],
        out_specs=[
            pl.BlockSpec((bm, dp), lambda m: (m, 0)),
            pl.BlockSpec((bm, dh), lambda m: (m, 0)),
            pl.BlockSpec((bm, dh), lambda m: (m, 0)),
        ],
        out_shape=[
            jax.ShapeDtypeStruct((n, dp), f32),
            jax.ShapeDtypeStruct((n, dh), f32),
            jax.ShapeDtypeStruct((n, dh), f32),
        ],
    )

    s0, s1 = conv_call(y0, y1, epack)
    z2, y0, y1 = mid_call(s0, s1, z1, dinv, bp[0], Wp[1])
    s0, s1 = conv_call(y0, y1, epack)
    z3, y0, y1 = mid_call(s0, s1, z2, dinv, bp[1], Wp[2])
    s0, s1 = conv_call(y0, y1, epack)

    out = pl.pallas_call(
        _fin_body,
        grid=(gm,),
        in_specs=[
            pl.BlockSpec((bm, dh), lambda m: (m, 0)),
            pl.BlockSpec((bm, dh), lambda m: (m, 0)),
            pl.BlockSpec((bm, dp), lambda m: (m, 0)),
            pl.BlockSpec((bm, 1), lambda m: (m, 0)),
            pl.BlockSpec((1, dp), lambda m: (0, 0)),
            pl.BlockSpec((dp, 1), lambda m: (0, 0)),
            pl.BlockSpec((1, 1), lambda m: (0, 0)),
        ],
        out_specs=pl.BlockSpec((bm, 1), lambda m: (m, 0)),
        out_shape=jax.ShapeDtypeStruct((n, 1), f32),
    )(s0, s1, z3, dinv, bp[2], W_out_p, b_out_p)

    return out


# speedup vs baseline: 9.0537x; 1.3377x over previous
import functools

import jax
import jax.numpy as jnp
from jax import lax
from jax.experimental import pallas as pl
from jax.experimental.pallas import tpu as pltpu
from jax.experimental.pallas import tpu_sc as plsc

CH = 128
DEGW = 128
NSUB = 16
NCORE = 2


def _round_up(x, m):
    return (x + m - 1) // m * m


def _deg_body(n_acc, chunks_per_tile, ep_ref, h0_ref, h1_ref,
              eidx, ones_v, zbuf, acc):
    c = lax.axis_index("c")
    s = lax.axis_index("s")
    w = s * NCORE + c

    def _memset(i, _):
        for j in range(DEGW // 16):
            ones_v[i, pl.ds(j * 16, 16)] = jnp.ones((16,), jnp.float32)
            zbuf[i, pl.ds(j * 16, 16)] = jnp.zeros((16,), jnp.float32)
        return 0
    lax.fori_loop(0, CH, _memset, 0)

    zc = n_acc // NSUB // CH
    for b in range(zc):
        pltpu.sync_copy(zbuf, acc.at[pl.ds(s * (n_acc // NSUB) + b * CH, CH)])
    plsc.subcore_barrier()

    def _body(i, _):
        ch = w * chunks_per_tile + i
        pltpu.sync_copy(ep_ref.at[ch], eidx)
        pltpu.sync_copy(ones_v, acc.at[eidx.at[1]], add=True)
        return 0
    lax.fori_loop(0, chunks_per_tile, _body, 0)
    plsc.subcore_barrier()

    rows = n_acc // NSUB
    off = s * rows

    @pl.when(c == 0)
    def _():
        pltpu.sync_copy(acc.at[pl.ds(off, rows)], h0_ref.at[pl.ds(off, rows)])

    @pl.when(c == 1)
    def _():
        pltpu.sync_copy(acc.at[pl.ds(off, rows)], h1_ref.at[pl.ds(off, rows)])


CCH = 64
R = 4
IR = 2 * R


def _conv_body(n_acc, dh, chunks_per_tile, y0_ref, y1_ref, ep_ref,
               o0_ref, o1_ref, idx_ring, bufs, isem, gsems, ssems, acc):
    c = lax.axis_index("c")
    s = lax.axis_index("s")

    def _memset(i, _):
        for j in range(dh // 16):
            bufs[0][i, pl.ds(j * 16, 16)] = jnp.zeros((16,), jnp.float32)
        return 0
    lax.fori_loop(0, CCH, _memset, 0)

    zrows = n_acc // NSUB
    zc = zrows // CCH
    for b in range(zc):
        pltpu.sync_copy(bufs[0], acc.at[pl.ds(s * zrows + b * CCH, CCH)])
    plsc.subcore_barrier()

    base = s * chunks_per_tile
    ngrp = chunks_per_tile // R

    def _run(y_ref):
        def _gather(k, b):
            return pltpu.async_copy(y_ref.at[idx_ring.at[k % IR, 0]], bufs[b],
                                    gsems[b])

        for k in range(IR):
            pltpu.async_copy(ep_ref.at[base + k], idx_ring.at[k], isem)
        for b in range(R):
            pltpu.make_async_copy(ep_ref.at[base + b], idx_ring.at[b],
                                  isem).wait()
            _gather(b, b)

        def _grp(g, _):
            scat = []
            for b in range(R):
                k = g * R + b
                pltpu.make_async_copy(y_ref.at[idx_ring.at[k % IR, 0]],
                                      bufs[b], gsems[b]).wait()
                scat.append(pltpu.async_copy(bufs[b],
                                             acc.at[idx_ring.at[k % IR, 1]],
                                             ssems[b], add=True))
            for b in range(R):
                k = g * R + b
                kn = k + R
                scat[b].wait()

                @pl.when(kn < chunks_per_tile)
                def _():
                    pltpu.make_async_copy(ep_ref.at[base + kn],
                                          idx_ring.at[kn % IR], isem).wait()
                    _gather(kn, b)

                    @pl.when(kn + R < chunks_per_tile)
                    def _():
                        pltpu.async_copy(ep_ref.at[base + kn + R],
                                         idx_ring.at[(kn + R) % IR], isem)
            return 0
        lax.fori_loop(0, ngrp, _grp, 0)

    @pl.when(c == 0)
    def _():
        _run(y0_ref)

    @pl.when(c == 1)
    def _():
        _run(y1_ref)

    plsc.subcore_barrier()

    rows = n_acc // NSUB
    off = s * rows

    @pl.when(c == 0)
    def _():
        pltpu.sync_copy(acc.at[pl.ds(off, rows)], o0_ref.at[pl.ds(off, rows)])

    @pl.when(c == 1)
    def _():
        pltpu.sync_copy(acc.at[pl.ds(off, rows)], o1_ref.at[pl.ds(off, rows)])


def _mm1_body(dh, x_ref, w_ref, b_ref, w1_ref, h0_ref, h1_ref,
              z_ref, y0_ref, y1_ref, dinv_ref, acc_ref):
    k = pl.program_id(1)

    @pl.when(k == 0)
    def _():
        acc_ref[...] = jnp.zeros_like(acc_ref)

    acc_ref[...] += jnp.dot(x_ref[...], w_ref[...],
                            preferred_element_type=jnp.float32)

    @pl.when(k == pl.num_programs(1) - 1)
    def _():
        h = jnp.maximum(acc_ref[...] + b_ref[...], 0.0)
        deg = 2.0 + h0_ref[:, 0:1] + h1_ref[:, 0:1]
        dinv = lax.rsqrt(deg)
        z = jnp.dot(h, w1_ref[...], preferred_element_type=jnp.float32)
        z_ref[...] = z
        y = z * dinv
        y0_ref[...] = y[:, :dh]
        y1_ref[...] = y[:, dh:]
        dinv_ref[...] = dinv


def _mid_body(dh, s0_ref, s1_ref, z_ref, dinv_ref, b_ref, w_ref,
              zo_ref, y0_ref, y1_ref):
    dinv = dinv_ref[...]
    sc = jnp.concatenate([s0_ref[...], s1_ref[...]], axis=1)
    x = dinv * sc + (2.0 * dinv * dinv) * z_ref[...] + b_ref[...]
    z = jnp.dot(x, w_ref[...], preferred_element_type=jnp.float32)
    zo_ref[...] = z
    y = z * dinv
    y0_ref[...] = y[:, :dh]
    y1_ref[...] = y[:, dh:]


def _fin_body(s0_ref, s1_ref, z_ref, dinv_ref, b_ref, wo_ref, bo_ref, o_ref):
    dinv = dinv_ref[...]
    sc = jnp.concatenate([s0_ref[...], s1_ref[...]], axis=1)
    x = dinv * sc + (2.0 * dinv * dinv) * z_ref[...] + b_ref[...]
    o_ref[...] = jax.nn.sigmoid(
        jnp.dot(x, wo_ref[...], preferred_element_type=jnp.float32) + bo_ref[...])


def kernel(PPI_x, PPI_edge_index, PPI_batch, edge_attr, W_in, b_in,
           W1, b1, W2, b2, W3, b3, W_out, b_out):
    del PPI_batch, edge_attr
    n, din = PPI_x.shape
    d = W1.shape[0]
    e = PPI_edge_index.shape[1]

    dp = _round_up(d, 256)
    dh = dp // 2
    n_acc = _round_up(n + CH, NSUB * CH)
    e_pad = _round_up(e, NSUB * CCH * R)

    pad = e_pad - e
    src = PPI_edge_index[0]
    dst = PPI_edge_index[1]
    src_p = jnp.concatenate([src, jnp.zeros((pad,), jnp.int32)])
    dst_p = jnp.concatenate([dst, jnp.full((pad,), n, jnp.int32)])
    sd = jnp.stack([src_p, dst_p])
    epack_d = sd.reshape(2, e_pad // CH, CH).transpose(1, 0, 2)
    epack_c = sd.reshape(2, e_pad // CCH, CCH).transpose(1, 0, 2)

    W_in_p = jnp.pad(W_in, ((0, 0), (0, dp - d)))
    b_in_p = jnp.pad(b_in, (0, dp - d)).reshape(1, dp)
    Wp = [jnp.pad(W, ((0, dp - d), (0, dp - d))) for W in (W1, W2, W3)]
    bp = [jnp.pad(b, (0, dp - d)).reshape(1, dp) for b in (b1, b2, b3)]
    W_out_p = jnp.pad(W_out, ((0, dp - d), (0, 0)))
    b_out_p = b_out.reshape(1, 1)

    f32 = jnp.float32
    mesh = plsc.VectorSubcoreMesh(core_axis_name="c", subcore_axis_name="s")

    deg_chunks = e_pad // (NCORE * NSUB * CH)
    deg_call = pl.kernel(
        functools.partial(_deg_body, n_acc, deg_chunks),
        out_type=[jax.ShapeDtypeStruct((n_acc, DEGW), f32)] * 2,
        mesh=mesh,
        scratch_types=[
            pltpu.VMEM((2, CH), jnp.int32),
            pltpu.VMEM((CH, DEGW), f32),
            pltpu.VMEM((CH, DEGW), f32),
            pltpu.VMEM_SHARED((n_acc, DEGW), f32),
        ],
    )
    h0, h1 = deg_call(epack_d)

    conv_chunks = e_pad // (NSUB * CCH)
    conv_call = pl.kernel(
        functools.partial(_conv_body, n_acc, dh, conv_chunks),
        out_type=[jax.ShapeDtypeStruct((n_acc, dh), f32)] * 2,
        mesh=mesh,
        scratch_types=[
            pltpu.VMEM((IR, 2, CCH), jnp.int32),
            [pltpu.VMEM((CCH, dh), f32) for _ in range(R)],
            pltpu.SemaphoreType.DMA,
            [pltpu.SemaphoreType.DMA for _ in range(R)],
            [pltpu.SemaphoreType.DMA for _ in range(R)],
            pltpu.VMEM_SHARED((n_acc, dh), f32),
        ],
    )

    bm = 1000
    bk = 1024
    gm, gk = n // bm, din // bk
    z1, y0, y1, dinv = pl.pallas_call(
        functools.partial(_mm1_body, dh),
        grid=(gm, gk),
        in_specs=[
            pl.BlockSpec((bm, bk), lambda m, k: (m, k)),
            pl.BlockSpec((bk, dp), lambda m, k: (k, 0)),
            pl.BlockSpec((1, dp), lambda m, k: (0, 0)),
            pl.BlockSpec((dp, dp), lambda m, k: (0, 0)),
            pl.BlockSpec((bm, DEGW), lambda m, k: (m, 0)),
            pl.BlockSpec((bm, DEGW), lambda m, k: (m, 0)),
        ],
        out_specs=[
            pl.BlockSpec((bm, dp), lambda m, k: (m, 0)),
            pl.BlockSpec((bm, dh), lambda m, k: (m, 0)),
            pl.BlockSpec((bm, dh), lambda m, k: (m, 0)),
            pl.BlockSpec((bm, 1), lambda m, k: (m, 0)),
        ],
        out_shape=[
            jax.ShapeDtypeStruct((n, dp), f32),
            jax.ShapeDtypeStruct((n, dh), f32),
            jax.ShapeDtypeStruct((n, dh), f32),
            jax.ShapeDtypeStruct((n, 1), f32),
        ],
        scratch_shapes=[pltpu.VMEM((bm, dp), f32)],
    )(PPI_x, W_in_p, b_in_p, Wp[0], h0, h1)

    mid_call = pl.pallas_call(
        functools.partial(_mid_body, dh),
        grid=(gm,),
        in_specs=[
            pl.BlockSpec((bm, dh), lambda m: (m, 0)),
            pl.BlockSpec((bm, dh), lambda m: (m, 0)),
            pl.BlockSpec((bm, dp), lambda m: (m, 0)),
            pl.BlockSpec((bm, 1), lambda m: (m, 0)),
            pl.BlockSpec((1, dp), lambda m: (0, 0)),
            pl.BlockSpec((dp, dp), lambda m: (0, 0)),
        ],
        out_specs=[
            pl.BlockSpec((bm, dp), lambda m: (m, 0)),
            pl.BlockSpec((bm, dh), lambda m: (m, 0)),
            pl.BlockSpec((bm, dh), lambda m: (m, 0)),
        ],
        out_shape=[
            jax.ShapeDtypeStruct((n, dp), f32),
            jax.ShapeDtypeStruct((n, dh), f32),
            jax.ShapeDtypeStruct((n, dh), f32),
        ],
    )

    s0, s1 = conv_call(y0, y1, epack_c)
    z2, y0, y1 = mid_call(s0, s1, z1, dinv, bp[0], Wp[1])
    s0, s1 = conv_call(y0, y1, epack_c)
    z3, y0, y1 = mid_call(s0, s1, z2, dinv, bp[1], Wp[2])
    s0, s1 = conv_call(y0, y1, epack_c)

    out = pl.pallas_call(
        _fin_body,
        grid=(gm,),
        in_specs=[
            pl.BlockSpec((bm, dh), lambda m: (m, 0)),
            pl.BlockSpec((bm, dh), lambda m: (m, 0)),
            pl.BlockSpec((bm, dp), lambda m: (m, 0)),
            pl.BlockSpec((bm, 1), lambda m: (m, 0)),
            pl.BlockSpec((1, dp), lambda m: (0, 0)),
            pl.BlockSpec((dp, 1), lambda m: (0, 0)),
            pl.BlockSpec((1, 1), lambda m: (0, 0)),
        ],
        out_specs=pl.BlockSpec((bm, 1), lambda m: (m, 0)),
        out_shape=jax.ShapeDtypeStruct((n, 1), f32),
    )(s0, s1, z3, dinv, bp[2], W_out_p, b_out_p)

    return out


# speedup vs baseline: 9.4947x; 1.0487x over previous
import functools

import jax
import jax.numpy as jnp
from jax import lax
from jax.experimental import pallas as pl
from jax.experimental.pallas import tpu as pltpu
from jax.experimental.pallas import tpu_sc as plsc

CH = 128
DEGW = 128
NSUB = 16
NCORE = 2


def _round_up(x, m):
    return (x + m - 1) // m * m


DIR = 8


def _deg_body(n_acc, chunks_per_tile, ep_ref, h0_ref, h1_ref,
              idx_ring, ones_v, zbuf, isem, ssem, acc):
    c = lax.axis_index("c")
    s = lax.axis_index("s")
    w = s * NCORE + c

    def _memset(i, _):
        for j in range(DEGW // 16):
            ones_v[i, pl.ds(j * 16, 16)] = jnp.ones((16,), jnp.float32)
            zbuf[i, pl.ds(j * 16, 16)] = jnp.zeros((16,), jnp.float32)
        return 0
    lax.fori_loop(0, CH, _memset, 0)

    zc = n_acc // NSUB // CH
    for b in range(zc):
        pltpu.sync_copy(zbuf, acc.at[pl.ds(s * (n_acc // NSUB) + b * CH, CH)])
    plsc.subcore_barrier()

    base = w * chunks_per_tile
    for k in range(4):
        pltpu.async_copy(ep_ref.at[base + k], idx_ring.at[k], isem)

    def _body(k, _):
        pltpu.make_async_copy(ep_ref.at[base], idx_ring.at[0], isem).wait()
        pltpu.async_copy(ones_v, acc.at[idx_ring.at[k % DIR, 1]], ssem,
                         add=True)

        @pl.when(k >= 3)
        def _():
            pltpu.make_async_copy(ones_v, acc.at[idx_ring.at[k % DIR, 1]],
                                  ssem).wait()

        @pl.when(k + 4 < chunks_per_tile)
        def _():
            pltpu.async_copy(ep_ref.at[base + k + 4],
                             idx_ring.at[(k + 4) % DIR], isem)
        return 0
    lax.fori_loop(0, chunks_per_tile, _body, 0)
    for _ in range(3):
        pltpu.make_async_copy(ones_v, acc.at[idx_ring.at[0, 1]], ssem).wait()
    plsc.subcore_barrier()

    rows = n_acc // NSUB
    off = s * rows

    @pl.when(c == 0)
    def _():
        pltpu.sync_copy(acc.at[pl.ds(off, rows)], h0_ref.at[pl.ds(off, rows)])

    @pl.when(c == 1)
    def _():
        pltpu.sync_copy(acc.at[pl.ds(off, rows)], h1_ref.at[pl.ds(off, rows)])


CCH = 64
R = 4
IR = 2 * R


def _conv_body(n_acc, dh, chunks_per_tile, y0_ref, y1_ref, ep_ref,
               o0_ref, o1_ref, idx_ring, bufs, isem, gsems, ssems, acc):
    c = lax.axis_index("c")
    s = lax.axis_index("s")

    def _memset(i, _):
        for j in range(dh // 16):
            bufs[0][i, pl.ds(j * 16, 16)] = jnp.zeros((16,), jnp.float32)
        return 0
    lax.fori_loop(0, CCH, _memset, 0)

    zrows = n_acc // NSUB
    zc = zrows // CCH
    for b in range(zc):
        pltpu.sync_copy(bufs[0], acc.at[pl.ds(s * zrows + b * CCH, CCH)])
    plsc.subcore_barrier()

    base = s * chunks_per_tile
    ngrp = chunks_per_tile // R

    def _run(y_ref):
        def _gather(k, b):
            return pltpu.async_copy(y_ref.at[idx_ring.at[k % IR, 0]], bufs[b],
                                    gsems[b])

        for k in range(IR):
            pltpu.async_copy(ep_ref.at[base + k], idx_ring.at[k], isem)
        for b in range(R):
            pltpu.make_async_copy(ep_ref.at[base + b], idx_ring.at[b],
                                  isem).wait()
            _gather(b, b)

        def _grp(g, _):
            scat = []
            for b in range(R):
                k = g * R + b
                pltpu.make_async_copy(y_ref.at[idx_ring.at[k % IR, 0]],
                                      bufs[b], gsems[b]).wait()
                scat.append(pltpu.async_copy(bufs[b],
                                             acc.at[idx_ring.at[k % IR, 1]],
                                             ssems[b], add=True))
            for b in range(R):
                k = g * R + b
                kn = k + R
                scat[b].wait()

                @pl.when(kn < chunks_per_tile)
                def _():
                    pltpu.make_async_copy(ep_ref.at[base + kn],
                                          idx_ring.at[kn % IR], isem).wait()
                    _gather(kn, b)

                    @pl.when(kn + R < chunks_per_tile)
                    def _():
                        pltpu.async_copy(ep_ref.at[base + kn + R],
                                         idx_ring.at[(kn + R) % IR], isem)
            return 0
        lax.fori_loop(0, ngrp, _grp, 0)

    @pl.when(c == 0)
    def _():
        _run(y0_ref)

    @pl.when(c == 1)
    def _():
        _run(y1_ref)

    plsc.subcore_barrier()

    rows = n_acc // NSUB
    off = s * rows

    @pl.when(c == 0)
    def _():
        pltpu.sync_copy(acc.at[pl.ds(off, rows)], o0_ref.at[pl.ds(off, rows)])

    @pl.when(c == 1)
    def _():
        pltpu.sync_copy(acc.at[pl.ds(off, rows)], o1_ref.at[pl.ds(off, rows)])


def _mma_body(x_ref, w_ref, b_ref, h_ref, acc_ref):
    k = pl.program_id(1)

    @pl.when(k == 0)
    def _():
        acc_ref[...] = jnp.zeros_like(acc_ref)

    acc_ref[...] += jnp.dot(x_ref[...], w_ref[...],
                            preferred_element_type=jnp.float32)

    @pl.when(k == pl.num_programs(1) - 1)
    def _():
        h_ref[...] = jnp.maximum(acc_ref[...] + b_ref[...], 0.0)


def _mmb_body(dh, h_ref, w1_ref, h0_ref, h1_ref,
              z_ref, y0_ref, y1_ref, dinv_ref):
    deg = 2.0 + h0_ref[:, 0:1] + h1_ref[:, 0:1]
    dinv = lax.rsqrt(deg)
    z = jnp.dot(h_ref[...], w1_ref[...], preferred_element_type=jnp.float32)
    z_ref[...] = z
    y = z * dinv
    y0_ref[...] = y[:, :dh]
    y1_ref[...] = y[:, dh:]
    dinv_ref[...] = dinv


def _mid_body(dh, s0_ref, s1_ref, z_ref, dinv_ref, b_ref, w_ref,
              zo_ref, y0_ref, y1_ref):
    dinv = dinv_ref[...]
    sc = jnp.concatenate([s0_ref[...], s1_ref[...]], axis=1)
    x = dinv * sc + (2.0 * dinv * dinv) * z_ref[...] + b_ref[...]
    z = jnp.dot(x, w_ref[...], preferred_element_type=jnp.float32)
    zo_ref[...] = z
    y = z * dinv
    y0_ref[...] = y[:, :dh]
    y1_ref[...] = y[:, dh:]


def _fin_body(s0_ref, s1_ref, z_ref, dinv_ref, b_ref, wo_ref, bo_ref, o_ref):
    dinv = dinv_ref[...]
    sc = jnp.concatenate([s0_ref[...], s1_ref[...]], axis=1)
    x = dinv * sc + (2.0 * dinv * dinv) * z_ref[...] + b_ref[...]
    o_ref[...] = jax.nn.sigmoid(
        jnp.dot(x, wo_ref[...], preferred_element_type=jnp.float32) + bo_ref[...])


def kernel(PPI_x, PPI_edge_index, PPI_batch, edge_attr, W_in, b_in,
           W1, b1, W2, b2, W3, b3, W_out, b_out):
    del PPI_batch, edge_attr
    n, din = PPI_x.shape
    d = W1.shape[0]
    e = PPI_edge_index.shape[1]

    dp = _round_up(d, 256)
    dh = dp // 2
    n_acc = _round_up(n + CH, NSUB * CH)
    e_pad = _round_up(e, NSUB * CCH * R)

    pad = e_pad - e
    src = PPI_edge_index[0]
    dst = PPI_edge_index[1]
    src_p = jnp.concatenate([src, jnp.zeros((pad,), jnp.int32)])
    dst_p = jnp.concatenate([dst, jnp.full((pad,), n, jnp.int32)])
    sd = jnp.stack([src_p, dst_p])
    epack_d = sd.reshape(2, e_pad // CH, CH).transpose(1, 0, 2)
    epack_c = sd.reshape(2, e_pad // CCH, CCH).transpose(1, 0, 2)

    W_in_p = jnp.pad(W_in, ((0, 0), (0, dp - d)))
    b_in_p = jnp.pad(b_in, (0, dp - d)).reshape(1, dp)
    Wp = [jnp.pad(W, ((0, dp - d), (0, dp - d))) for W in (W1, W2, W3)]
    bp = [jnp.pad(b, (0, dp - d)).reshape(1, dp) for b in (b1, b2, b3)]
    W_out_p = jnp.pad(W_out, ((0, dp - d), (0, 0)))
    b_out_p = b_out.reshape(1, 1)

    f32 = jnp.float32
    mesh = plsc.VectorSubcoreMesh(core_axis_name="c", subcore_axis_name="s")

    deg_chunks = e_pad // (NCORE * NSUB * CH)
    deg_call = pl.kernel(
        functools.partial(_deg_body, n_acc, deg_chunks),
        out_type=[jax.ShapeDtypeStruct((n_acc, DEGW), f32)] * 2,
        mesh=mesh,
        scratch_types=[
            pltpu.VMEM((DIR, 2, CH), jnp.int32),
            pltpu.VMEM((CH, DEGW), f32),
            pltpu.VMEM((CH, DEGW), f32),
            pltpu.SemaphoreType.DMA,
            pltpu.SemaphoreType.DMA,
            pltpu.VMEM_SHARED((n_acc, DEGW), f32),
        ],
    )
    h0, h1 = deg_call(epack_d)

    conv_chunks = e_pad // (NSUB * CCH)
    conv_call = pl.kernel(
        functools.partial(_conv_body, n_acc, dh, conv_chunks),
        out_type=[jax.ShapeDtypeStruct((n_acc, dh), f32)] * 2,
        mesh=mesh,
        scratch_types=[
            pltpu.VMEM((IR, 2, CCH), jnp.int32),
            [pltpu.VMEM((CCH, dh), f32) for _ in range(R)],
            pltpu.SemaphoreType.DMA,
            [pltpu.SemaphoreType.DMA for _ in range(R)],
            [pltpu.SemaphoreType.DMA for _ in range(R)],
            pltpu.VMEM_SHARED((n_acc, dh), f32),
        ],
    )

    bm = 1000
    bk = 1024
    gm, gk = n // bm, din // bk
    hmat = pl.pallas_call(
        _mma_body,
        grid=(gm, gk),
        in_specs=[
            pl.BlockSpec((bm, bk), lambda m, k: (m, k)),
            pl.BlockSpec((bk, dp), lambda m, k: (k, 0)),
            pl.BlockSpec((1, dp), lambda m, k: (0, 0)),
        ],
        out_specs=pl.BlockSpec((bm, dp), lambda m, k: (m, 0)),
        out_shape=jax.ShapeDtypeStruct((n, dp), f32),
        scratch_shapes=[pltpu.VMEM((bm, dp), f32)],
    )(PPI_x, W_in_p, b_in_p)

    z1, y0, y1, dinv = pl.pallas_call(
        functools.partial(_mmb_body, dh),
        grid=(gm,),
        in_specs=[
            pl.BlockSpec((bm, dp), lambda m: (m, 0)),
            pl.BlockSpec((dp, dp), lambda m: (0, 0)),
            pl.BlockSpec((bm, DEGW), lambda m: (m, 0)),
            pl.BlockSpec((bm, DEGW), lambda m: (m, 0)),
        ],
        out_specs=[
            pl.BlockSpec((bm, dp), lambda m: (m, 0)),
            pl.BlockSpec((bm, dh), lambda m: (m, 0)),
            pl.BlockSpec((bm, dh), lambda m: (m, 0)),
            pl.BlockSpec((bm, 1), lambda m: (m, 0)),
        ],
        out_shape=[
            jax.ShapeDtypeStruct((n, dp), f32),
            jax.ShapeDtypeStruct((n, dh), f32),
            jax.ShapeDtypeStruct((n, dh), f32),
            jax.ShapeDtypeStruct((n, 1), f32),
        ],
    )(hmat, Wp[0], h0, h1)

    mid_call = pl.pallas_call(
        functools.partial(_mid_body, dh),
        grid=(gm,),
        in_specs=[
            pl.BlockSpec((bm, dh), lambda m: (m, 0)),
            pl.BlockSpec((bm, dh), lambda m: (m, 0)),
            pl.BlockSpec((bm, dp), lambda m: (m, 0)),
            pl.BlockSpec((bm, 1), lambda m: (m, 0)),
            pl.BlockSpec((1, dp), lambda m: (0, 0)),
            pl.BlockSpec((dp, dp), lambda m: (0, 0)),
        ],
        out_specs=[
            pl.BlockSpec((bm, dp), lambda m: (m, 0)),
            pl.BlockSpec((bm, dh), lambda m: (m, 0)),
            pl.BlockSpec((bm, dh), lambda m: (m, 0)),
        ],
        out_shape=[
            jax.ShapeDtypeStruct((n, dp), f32),
            jax.ShapeDtypeStruct((n, dh), f32),
            jax.ShapeDtypeStruct((n, dh), f32),
        ],
    )

    s0, s1 = conv_call(y0, y1, epack_c)
    z2, y0, y1 = mid_call(s0, s1, z1, dinv, bp[0], Wp[1])
    s0, s1 = conv_call(y0, y1, epack_c)
    z3, y0, y1 = mid_call(s0, s1, z2, dinv, bp[1], Wp[2])
    s0, s1 = conv_call(y0, y1, epack_c)

    out = pl.pallas_call(
        _fin_body,
        grid=(gm,),
        in_specs=[
            pl.BlockSpec((bm, dh), lambda m: (m, 0)),
            pl.BlockSpec((bm, dh), lambda m: (m, 0)),
            pl.BlockSpec((bm, dp), lambda m: (m, 0)),
            pl.BlockSpec((bm, 1), lambda m: (m, 0)),
            pl.BlockSpec((1, dp), lambda m: (0, 0)),
            pl.BlockSpec((dp, 1), lambda m: (0, 0)),
            pl.BlockSpec((1, 1), lambda m: (0, 0)),
        ],
        out_specs=pl.BlockSpec((bm, 1), lambda m: (m, 0)),
        out_shape=jax.ShapeDtypeStruct((n, 1), f32),
    )(s0, s1, z3, dinv, bp[2], W_out_p, b_out_p)

    return out


# speedup vs baseline: 10.4091x; 1.0963x over previous
import functools

import jax
import jax.numpy as jnp
from jax import lax
from jax.experimental import pallas as pl
from jax.experimental.pallas import tpu as pltpu
from jax.experimental.pallas import tpu_sc as plsc

CH = 128
DEGW = 128
NSUB = 16
NCORE = 2


def _round_up(x, m):
    return (x + m - 1) // m * m


DIR = 8


def _deg_body(n_acc, chunks_per_tile, ep_ref, h0_ref, h1_ref,
              idx_ring, ones_v, zbuf, isem, ssem, acc):
    c = lax.axis_index("c")
    s = lax.axis_index("s")
    w = s * NCORE + c

    def _memset(i, _):
        for j in range(DEGW // 16):
            ones_v[i, pl.ds(j * 16, 16)] = jnp.ones((16,), jnp.float32)
            zbuf[i, pl.ds(j * 16, 16)] = jnp.zeros((16,), jnp.float32)
        return 0
    lax.fori_loop(0, CH, _memset, 0)

    zc = n_acc // NSUB // CH
    for b in range(zc):
        pltpu.sync_copy(zbuf, acc.at[pl.ds(s * (n_acc // NSUB) + b * CH, CH)])
    plsc.subcore_barrier()

    base = w * chunks_per_tile
    for k in range(4):
        pltpu.async_copy(ep_ref.at[base + k], idx_ring.at[k], isem)

    def _body(k, _):
        pltpu.make_async_copy(ep_ref.at[base], idx_ring.at[0], isem).wait()
        pltpu.async_copy(ones_v, acc.at[idx_ring.at[k % DIR, 1]], ssem,
                         add=True)

        @pl.when(k >= 3)
        def _():
            pltpu.make_async_copy(ones_v, acc.at[idx_ring.at[k % DIR, 1]],
                                  ssem).wait()

        @pl.when(k + 4 < chunks_per_tile)
        def _():
            pltpu.async_copy(ep_ref.at[base + k + 4],
                             idx_ring.at[(k + 4) % DIR], isem)
        return 0
    lax.fori_loop(0, chunks_per_tile, _body, 0)
    for _ in range(3):
        pltpu.make_async_copy(ones_v, acc.at[idx_ring.at[0, 1]], ssem).wait()
    plsc.subcore_barrier()

    rows = n_acc // NSUB
    off = s * rows

    @pl.when(c == 0)
    def _():
        pltpu.sync_copy(acc.at[pl.ds(off, rows)], h0_ref.at[pl.ds(off, rows)])

    @pl.when(c == 1)
    def _():
        pltpu.sync_copy(acc.at[pl.ds(off, rows)], h1_ref.at[pl.ds(off, rows)])


CCH = 32
R = 8
IR = 2 * R


def _conv_body(n_acc, dh, chunks_per_tile, y0_ref, y1_ref, ep_ref,
               o0_ref, o1_ref, idx_ring, bufs, isem, gsems, ssems, acc):
    c = lax.axis_index("c")
    s = lax.axis_index("s")

    def _memset(i, _):
        for j in range(dh // 16):
            bufs[0][i, pl.ds(j * 16, 16)] = jnp.zeros((16,), jnp.float32)
        return 0
    lax.fori_loop(0, CCH, _memset, 0)

    zrows = n_acc // NSUB
    zc = zrows // CCH
    for b in range(zc):
        pltpu.sync_copy(bufs[0], acc.at[pl.ds(s * zrows + b * CCH, CCH)])
    plsc.subcore_barrier()

    base = s * chunks_per_tile
    ngrp = chunks_per_tile // R

    def _run(y_ref):
        def _gather(k, b):
            return pltpu.async_copy(y_ref.at[idx_ring.at[k % IR, 0]], bufs[b],
                                    gsems[b])

        for k in range(IR):
            pltpu.async_copy(ep_ref.at[base + k], idx_ring.at[k], isem)
        for b in range(R):
            pltpu.make_async_copy(ep_ref.at[base + b], idx_ring.at[b],
                                  isem).wait()
            _gather(b, b)

        def _grp(g, _):
            scat = []
            for b in range(R):
                k = g * R + b
                pltpu.make_async_copy(y_ref.at[idx_ring.at[k % IR, 0]],
                                      bufs[b], gsems[b]).wait()
                scat.append(pltpu.async_copy(bufs[b],
                                             acc.at[idx_ring.at[k % IR, 1]],
                                             ssems[b], add=True))
            for b in range(R):
                k = g * R + b
                kn = k + R
                scat[b].wait()

                @pl.when(kn < chunks_per_tile)
                def _():
                    pltpu.make_async_copy(ep_ref.at[base + kn],
                                          idx_ring.at[kn % IR], isem).wait()
                    _gather(kn, b)

                    @pl.when(kn + R < chunks_per_tile)
                    def _():
                        pltpu.async_copy(ep_ref.at[base + kn + R],
                                         idx_ring.at[(kn + R) % IR], isem)
            return 0
        lax.fori_loop(0, ngrp, _grp, 0)

    @pl.when(c == 0)
    def _():
        _run(y0_ref)

    @pl.when(c == 1)
    def _():
        _run(y1_ref)

    plsc.subcore_barrier()

    rows = n_acc // NSUB
    off = s * rows

    @pl.when(c == 0)
    def _():
        pltpu.sync_copy(acc.at[pl.ds(off, rows)], o0_ref.at[pl.ds(off, rows)])

    @pl.when(c == 1)
    def _():
        pltpu.sync_copy(acc.at[pl.ds(off, rows)], o1_ref.at[pl.ds(off, rows)])


def _mma_body(x_ref, w_ref, b_ref, h_ref, acc_ref):
    k = pl.program_id(1)

    @pl.when(k == 0)
    def _():
        acc_ref[...] = jnp.zeros_like(acc_ref)

    acc_ref[...] += jnp.dot(x_ref[...].astype(jnp.bfloat16),
                            w_ref[...].astype(jnp.bfloat16),
                            preferred_element_type=jnp.float32)

    @pl.when(k == pl.num_programs(1) - 1)
    def _():
        h_ref[...] = jnp.maximum(acc_ref[...] + b_ref[...], 0.0)


def _mmb_body(dh, h_ref, w1_ref, h0_ref, h1_ref,
              z_ref, y0_ref, y1_ref, dinv_ref):
    deg = 2.0 + h0_ref[:, 0:1] + h1_ref[:, 0:1]
    dinv = lax.rsqrt(deg)
    z = jnp.dot(h_ref[...], w1_ref[...], preferred_element_type=jnp.float32)
    z_ref[...] = z
    y = z * dinv
    y0_ref[...] = y[:, :dh]
    y1_ref[...] = y[:, dh:]
    dinv_ref[...] = dinv


def _mid_body(dh, s0_ref, s1_ref, z_ref, dinv_ref, b_ref, w_ref,
              zo_ref, y0_ref, y1_ref):
    dinv = dinv_ref[...]
    sc = jnp.concatenate([s0_ref[...], s1_ref[...]], axis=1)
    x = dinv * sc + (2.0 * dinv * dinv) * z_ref[...] + b_ref[...]
    z = jnp.dot(x, w_ref[...], preferred_element_type=jnp.float32)
    zo_ref[...] = z
    y = z * dinv
    y0_ref[...] = y[:, :dh]
    y1_ref[...] = y[:, dh:]


def _fin_body(s0_ref, s1_ref, z_ref, dinv_ref, b_ref, wo_ref, bo_ref, o_ref):
    dinv = dinv_ref[...]
    sc = jnp.concatenate([s0_ref[...], s1_ref[...]], axis=1)
    x = dinv * sc + (2.0 * dinv * dinv) * z_ref[...] + b_ref[...]
    o_ref[...] = jax.nn.sigmoid(
        jnp.dot(x, wo_ref[...], preferred_element_type=jnp.float32) + bo_ref[...])


def kernel(PPI_x, PPI_edge_index, PPI_batch, edge_attr, W_in, b_in,
           W1, b1, W2, b2, W3, b3, W_out, b_out):
    del PPI_batch, edge_attr
    n, din = PPI_x.shape
    d = W1.shape[0]
    e = PPI_edge_index.shape[1]

    dp = _round_up(d, 256)
    dh = dp // 2
    n_acc = _round_up(n + CH, NSUB * CH)
    e_pad = _round_up(e, NSUB * CCH * R)

    pad = e_pad - e
    src = PPI_edge_index[0]
    dst = PPI_edge_index[1]
    src_p = jnp.concatenate([src, jnp.zeros((pad,), jnp.int32)])
    dst_p = jnp.concatenate([dst, jnp.full((pad,), n, jnp.int32)])
    sd = jnp.stack([src_p, dst_p])
    epack_d = sd.reshape(2, e_pad // CH, CH).transpose(1, 0, 2)
    epack_c = sd.reshape(2, e_pad // CCH, CCH).transpose(1, 0, 2)

    W_in_p = jnp.pad(W_in, ((0, 0), (0, dp - d)))
    b_in_p = jnp.pad(b_in, (0, dp - d)).reshape(1, dp)
    Wp = [jnp.pad(W, ((0, dp - d), (0, dp - d))) for W in (W1, W2, W3)]
    bp = [jnp.pad(b, (0, dp - d)).reshape(1, dp) for b in (b1, b2, b3)]
    W_out_p = jnp.pad(W_out, ((0, dp - d), (0, 0)))
    b_out_p = b_out.reshape(1, 1)

    f32 = jnp.float32
    mesh = plsc.VectorSubcoreMesh(core_axis_name="c", subcore_axis_name="s")

    deg_chunks = e_pad // (NCORE * NSUB * CH)
    deg_call = pl.kernel(
        functools.partial(_deg_body, n_acc, deg_chunks),
        out_type=[jax.ShapeDtypeStruct((n_acc, DEGW), f32)] * 2,
        mesh=mesh,
        scratch_types=[
            pltpu.VMEM((DIR, 2, CH), jnp.int32),
            pltpu.VMEM((CH, DEGW), f32),
            pltpu.VMEM((CH, DEGW), f32),
            pltpu.SemaphoreType.DMA,
            pltpu.SemaphoreType.DMA,
            pltpu.VMEM_SHARED((n_acc, DEGW), f32),
        ],
    )
    h0, h1 = deg_call(epack_d)

    conv_chunks = e_pad // (NSUB * CCH)
    conv_call = pl.kernel(
        functools.partial(_conv_body, n_acc, dh, conv_chunks),
        out_type=[jax.ShapeDtypeStruct((n_acc, dh), f32)] * 2,
        mesh=mesh,
        scratch_types=[
            pltpu.VMEM((IR, 2, CCH), jnp.int32),
            [pltpu.VMEM((CCH, dh), f32) for _ in range(R)],
            pltpu.SemaphoreType.DMA,
            [pltpu.SemaphoreType.DMA for _ in range(R)],
            [pltpu.SemaphoreType.DMA for _ in range(R)],
            pltpu.VMEM_SHARED((n_acc, dh), f32),
        ],
    )

    bm = 2000
    bk = 1024
    gm, gk = n // bm, din // bk
    hmat = pl.pallas_call(
        _mma_body,
        grid=(gm, gk),
        in_specs=[
            pl.BlockSpec((bm, bk), lambda m, k: (m, k)),
            pl.BlockSpec((bk, dp), lambda m, k: (k, 0)),
            pl.BlockSpec((1, dp), lambda m, k: (0, 0)),
        ],
        out_specs=pl.BlockSpec((bm, dp), lambda m, k: (m, 0)),
        out_shape=jax.ShapeDtypeStruct((n, dp), f32),
        scratch_shapes=[pltpu.VMEM((bm, dp), f32)],
    )(PPI_x, W_in_p, b_in_p)

    z1, y0, y1, dinv = pl.pallas_call(
        functools.partial(_mmb_body, dh),
        grid=(gm,),
        in_specs=[
            pl.BlockSpec((bm, dp), lambda m: (m, 0)),
            pl.BlockSpec((dp, dp), lambda m: (0, 0)),
            pl.BlockSpec((bm, DEGW), lambda m: (m, 0)),
            pl.BlockSpec((bm, DEGW), lambda m: (m, 0)),
        ],
        out_specs=[
            pl.BlockSpec((bm, dp), lambda m: (m, 0)),
            pl.BlockSpec((bm, dh), lambda m: (m, 0)),
            pl.BlockSpec((bm, dh), lambda m: (m, 0)),
            pl.BlockSpec((bm, 1), lambda m: (m, 0)),
        ],
        out_shape=[
            jax.ShapeDtypeStruct((n, dp), f32),
            jax.ShapeDtypeStruct((n, dh), f32),
            jax.ShapeDtypeStruct((n, dh), f32),
            jax.ShapeDtypeStruct((n, 1), f32),
        ],
    )(hmat, Wp[0], h0, h1)

    mid_call = pl.pallas_call(
        functools.partial(_mid_body, dh),
        grid=(gm,),
        in_specs=[
            pl.BlockSpec((bm, dh), lambda m: (m, 0)),
            pl.BlockSpec((bm, dh), lambda m: (m, 0)),
            pl.BlockSpec((bm, dp), lambda m: (m, 0)),
            pl.BlockSpec((bm, 1), lambda m: (m, 0)),
            pl.BlockSpec((1, dp), lambda m: (0, 0)),
            pl.BlockSpec((dp, dp), lambda m: (0, 0)),
        ],
        out_specs=[
            pl.BlockSpec((bm, dp), lambda m: (m, 0)),
            pl.BlockSpec((bm, dh), lambda m: (m, 0)),
            pl.BlockSpec((bm, dh), lambda m: (m, 0)),
        ],
        out_shape=[
            jax.ShapeDtypeStruct((n, dp), f32),
            jax.ShapeDtypeStruct((n, dh), f32),
            jax.ShapeDtypeStruct((n, dh), f32),
        ],
    )

    s0, s1 = conv_call(y0, y1, epack_c)
    z2, y0, y1 = mid_call(s0, s1, z1, dinv, bp[0], Wp[1])
    s0, s1 = conv_call(y0, y1, epack_c)
    z3, y0, y1 = mid_call(s0, s1, z2, dinv, bp[1], Wp[2])
    s0, s1 = conv_call(y0, y1, epack_c)

    out = pl.pallas_call(
        _fin_body,
        grid=(gm,),
        in_specs=[
            pl.BlockSpec((bm, dh), lambda m: (m, 0)),
            pl.BlockSpec((bm, dh), lambda m: (m, 0)),
            pl.BlockSpec((bm, dp), lambda m: (m, 0)),
            pl.BlockSpec((bm, 1), lambda m: (m, 0)),
            pl.BlockSpec((1, dp), lambda m: (0, 0)),
            pl.BlockSpec((dp, 1), lambda m: (0, 0)),
            pl.BlockSpec((1, 1), lambda m: (0, 0)),
        ],
        out_specs=pl.BlockSpec((bm, 1), lambda m: (m, 0)),
        out_shape=jax.ShapeDtypeStruct((n, 1), f32),
    )(s0, s1, z3, dinv, bp[2], W_out_p, b_out_p)

    return out


# speedup vs baseline: 11.8450x; 1.1379x over previous
import functools

import jax
import jax.numpy as jnp
from jax import lax
from jax.experimental import pallas as pl
from jax.experimental.pallas import tpu as pltpu
from jax.experimental.pallas import tpu_sc as plsc

CH = 128
DEGW = 128
NSUB = 16
NCORE = 2


def _round_up(x, m):
    return (x + m - 1) // m * m


DIR = 8


def _deg_body(n_acc, chunks_per_tile, ep_ref, h0_ref, h1_ref,
              idx_ring, ones_v, zbuf, isem, ssem, acc):
    c = lax.axis_index("c")
    s = lax.axis_index("s")
    w = s * NCORE + c

    def _memset(i, _):
        for j in range(DEGW // 16):
            ones_v[i, pl.ds(j * 16, 16)] = jnp.ones((16,), jnp.float32)
            zbuf[i, pl.ds(j * 16, 16)] = jnp.zeros((16,), jnp.float32)
        return 0
    lax.fori_loop(0, CH, _memset, 0)

    zc = n_acc // NSUB // CH
    for b in range(zc):
        pltpu.sync_copy(zbuf, acc.at[pl.ds(s * (n_acc // NSUB) + b * CH, CH)])
    plsc.subcore_barrier()

    base = w * chunks_per_tile
    for k in range(4):
        pltpu.async_copy(ep_ref.at[base + k], idx_ring.at[k], isem)

    def _body(k, _):
        pltpu.make_async_copy(ep_ref.at[base], idx_ring.at[0], isem).wait()
        pltpu.async_copy(ones_v, acc.at[idx_ring.at[k % DIR, 1]], ssem,
                         add=True)

        @pl.when(k >= 3)
        def _():
            pltpu.make_async_copy(ones_v, acc.at[idx_ring.at[k % DIR, 1]],
                                  ssem).wait()

        @pl.when(k + 4 < chunks_per_tile)
        def _():
            pltpu.async_copy(ep_ref.at[base + k + 4],
                             idx_ring.at[(k + 4) % DIR], isem)
        return 0
    lax.fori_loop(0, chunks_per_tile, _body, 0)
    for _ in range(3):
        pltpu.make_async_copy(ones_v, acc.at[idx_ring.at[0, 1]], ssem).wait()
    plsc.subcore_barrier()

    rows = n_acc // NSUB
    off = s * rows

    @pl.when(c == 0)
    def _():
        pltpu.sync_copy(acc.at[pl.ds(off, rows)], h0_ref.at[pl.ds(off, rows)])

    @pl.when(c == 1)
    def _():
        pltpu.sync_copy(acc.at[pl.ds(off, rows)], h1_ref.at[pl.ds(off, rows)])


CCH = 32
R = 10
IR = 2 * R


def _conv_body(n_acc, dh, chunks_per_tile, y0_ref, y1_ref, ep_ref,
               o0_ref, o1_ref, idx_ring, bufs, isem, gsems, ssems, acc):
    c = lax.axis_index("c")
    s = lax.axis_index("s")

    def _memset(i, _):
        for j in range(dh // 16):
            bufs[0][i, pl.ds(j * 16, 16)] = jnp.zeros((16,), jnp.float32)
        return 0
    lax.fori_loop(0, CCH, _memset, 0)

    zrows = n_acc // NSUB
    zc = zrows // CCH
    for b in range(zc):
        pltpu.sync_copy(bufs[0], acc.at[pl.ds(s * zrows + b * CCH, CCH)])
    plsc.subcore_barrier()

    base = s * chunks_per_tile
    ngrp = chunks_per_tile // R

    def _run(y_ref):
        def _gather(k, b):
            return pltpu.async_copy(y_ref.at[idx_ring.at[k % IR, 0]], bufs[b],
                                    gsems[b])

        for k in range(IR):
            pltpu.async_copy(ep_ref.at[base + k], idx_ring.at[k], isem)
        for b in range(R):
            pltpu.make_async_copy(ep_ref.at[base + b], idx_ring.at[b],
                                  isem).wait()
            _gather(b, b)

        def _grp(g, _):
            scat = []
            for b in range(R):
                k = g * R + b
                pltpu.make_async_copy(y_ref.at[idx_ring.at[k % IR, 0]],
                                      bufs[b], gsems[b]).wait()
                scat.append(pltpu.async_copy(bufs[b],
                                             acc.at[idx_ring.at[k % IR, 1]],
                                             ssems[b], add=True))
            for b in range(R):
                k = g * R + b
                kn = k + R
                scat[b].wait()

                @pl.when(kn < chunks_per_tile)
                def _():
                    pltpu.make_async_copy(ep_ref.at[base + kn],
                                          idx_ring.at[kn % IR], isem).wait()
                    _gather(kn, b)

                    @pl.when(kn + R < chunks_per_tile)
                    def _():
                        pltpu.async_copy(ep_ref.at[base + kn + R],
                                         idx_ring.at[(kn + R) % IR], isem)
            return 0
        lax.fori_loop(0, ngrp, _grp, 0)

    @pl.when(c == 0)
    def _():
        _run(y0_ref)

    @pl.when(c == 1)
    def _():
        _run(y1_ref)

    plsc.subcore_barrier()

    rows = n_acc // NSUB
    off = s * rows

    @pl.when(c == 0)
    def _():
        pltpu.sync_copy(acc.at[pl.ds(off, rows)], o0_ref.at[pl.ds(off, rows)])

    @pl.when(c == 1)
    def _():
        pltpu.sync_copy(acc.at[pl.ds(off, rows)], o1_ref.at[pl.ds(off, rows)])


def _mma_body(x_ref, w_ref, b_ref, h_ref, acc_ref):
    k = pl.program_id(1)

    @pl.when(k == 0)
    def _():
        acc_ref[...] = jnp.zeros_like(acc_ref)

    acc_ref[...] += jnp.dot(x_ref[...].astype(jnp.bfloat16),
                            w_ref[...].astype(jnp.bfloat16),
                            preferred_element_type=jnp.float32)

    @pl.when(k == pl.num_programs(1) - 1)
    def _():
        h_ref[...] = jnp.maximum(acc_ref[...] + b_ref[...], 0.0)


def _mmb_body(dh, h_ref, w1_ref, h0_ref, h1_ref,
              z_ref, y0_ref, y1_ref, dinv_ref):
    deg = 2.0 + h0_ref[:, 0:1] + h1_ref[:, 0:1]
    dinv = lax.rsqrt(deg)
    z = jnp.dot(h_ref[...], w1_ref[...], preferred_element_type=jnp.float32)
    z_ref[...] = z
    y = z * dinv
    y0_ref[...] = y[:, :dh]
    y1_ref[...] = y[:, dh:]
    dinv_ref[...] = dinv


def _mid_body(dh, s0_ref, s1_ref, z_ref, dinv_ref, b_ref, w_ref,
              zo_ref, y0_ref, y1_ref):
    dinv = dinv_ref[...]
    sc = jnp.concatenate([s0_ref[...], s1_ref[...]], axis=1)
    x = dinv * sc + (2.0 * dinv * dinv) * z_ref[...] + b_ref[...]
    z = jnp.dot(x, w_ref[...], preferred_element_type=jnp.float32)
    zo_ref[...] = z
    y = z * dinv
    y0_ref[...] = y[:, :dh]
    y1_ref[...] = y[:, dh:]


def _fin_body(s0_ref, s1_ref, z_ref, dinv_ref, b_ref, wo_ref, bo_ref, o_ref):
    dinv = dinv_ref[...]
    sc = jnp.concatenate([s0_ref[...], s1_ref[...]], axis=1)
    x = dinv * sc + (2.0 * dinv * dinv) * z_ref[...] + b_ref[...]
    o_ref[...] = jax.nn.sigmoid(
        jnp.dot(x, wo_ref[...], preferred_element_type=jnp.float32) + bo_ref[...])


def kernel(PPI_x, PPI_edge_index, PPI_batch, edge_attr, W_in, b_in,
           W1, b1, W2, b2, W3, b3, W_out, b_out):
    del PPI_batch, edge_attr
    n, din = PPI_x.shape
    d = W1.shape[0]
    e = PPI_edge_index.shape[1]

    dp = _round_up(d, 256)
    dh = dp // 2
    n_acc = _round_up(n + CH, NSUB * CH)
    e_pad = _round_up(e, NSUB * CCH * R)

    pad = e_pad - e
    src = PPI_edge_index[0]
    dst = PPI_edge_index[1]
    src_p = jnp.concatenate([src, jnp.zeros((pad,), jnp.int32)])
    dst_p = jnp.concatenate([dst, jnp.full((pad,), n, jnp.int32)])
    sd = jnp.stack([src_p, dst_p])
    epack_d = sd.reshape(2, e_pad // CH, CH).transpose(1, 0, 2)
    epack_c = sd.reshape(2, e_pad // CCH, CCH).transpose(1, 0, 2)

    W_in_p = jnp.pad(W_in, ((0, 0), (0, dp - d)))
    b_in_p = jnp.pad(b_in, (0, dp - d)).reshape(1, dp)
    Wp = [jnp.pad(W, ((0, dp - d), (0, dp - d))) for W in (W1, W2, W3)]
    bp = [jnp.pad(b, (0, dp - d)).reshape(1, dp) for b in (b1, b2, b3)]
    W_out_p = jnp.pad(W_out, ((0, dp - d), (0, 0)))
    b_out_p = b_out.reshape(1, 1)

    f32 = jnp.float32
    mesh = plsc.VectorSubcoreMesh(core_axis_name="c", subcore_axis_name="s")

    deg_chunks = e_pad // (NCORE * NSUB * CH)
    deg_call = pl.kernel(
        functools.partial(_deg_body, n_acc, deg_chunks),
        out_type=[jax.ShapeDtypeStruct((n_acc, DEGW), f32)] * 2,
        mesh=mesh,
        scratch_types=[
            pltpu.VMEM((DIR, 2, CH), jnp.int32),
            pltpu.VMEM((CH, DEGW), f32),
            pltpu.VMEM((CH, DEGW), f32),
            pltpu.SemaphoreType.DMA,
            pltpu.SemaphoreType.DMA,
            pltpu.VMEM_SHARED((n_acc, DEGW), f32),
        ],
    )
    h0, h1 = deg_call(epack_d)

    conv_chunks = e_pad // (NSUB * CCH)
    conv_call = pl.kernel(
        functools.partial(_conv_body, n_acc, dh, conv_chunks),
        out_type=[jax.ShapeDtypeStruct((n_acc, dh), f32)] * 2,
        mesh=mesh,
        scratch_types=[
            pltpu.VMEM((IR, 2, CCH), jnp.int32),
            [pltpu.VMEM((CCH, dh), f32) for _ in range(R)],
            pltpu.SemaphoreType.DMA,
            [pltpu.SemaphoreType.DMA for _ in range(R)],
            [pltpu.SemaphoreType.DMA for _ in range(R)],
            pltpu.VMEM_SHARED((n_acc, dh), f32),
        ],
    )

    bm = 2000
    bk = 1024
    gm, gk = n // bm, din // bk
    hmat = pl.pallas_call(
        _mma_body,
        grid=(gm, gk),
        in_specs=[
            pl.BlockSpec((bm, bk), lambda m, k: (m, k)),
            pl.BlockSpec((bk, dp), lambda m, k: (k, 0)),
            pl.BlockSpec((1, dp), lambda m, k: (0, 0)),
        ],
        out_specs=pl.BlockSpec((bm, dp), lambda m, k: (m, 0)),
        out_shape=jax.ShapeDtypeStruct((n, dp), f32),
        scratch_shapes=[pltpu.VMEM((bm, dp), f32)],
    )(PPI_x, W_in_p, b_in_p)

    z1, y0, y1, dinv = pl.pallas_call(
        functools.partial(_mmb_body, dh),
        grid=(gm,),
        in_specs=[
            pl.BlockSpec((bm, dp), lambda m: (m, 0)),
            pl.BlockSpec((dp, dp), lambda m: (0, 0)),
            pl.BlockSpec((bm, DEGW), lambda m: (m, 0)),
            pl.BlockSpec((bm, DEGW), lambda m: (m, 0)),
        ],
        out_specs=[
            pl.BlockSpec((bm, dp), lambda m: (m, 0)),
            pl.BlockSpec((bm, dh), lambda m: (m, 0)),
            pl.BlockSpec((bm, dh), lambda m: (m, 0)),
            pl.BlockSpec((bm, 1), lambda m: (m, 0)),
        ],
        out_shape=[
            jax.ShapeDtypeStruct((n, dp), f32),
            jax.ShapeDtypeStruct((n, dh), f32),
            jax.ShapeDtypeStruct((n, dh), f32),
            jax.ShapeDtypeStruct((n, 1), f32),
        ],
    )(hmat, Wp[0], h0, h1)

    mid_call = pl.pallas_call(
        functools.partial(_mid_body, dh),
        grid=(gm,),
        in_specs=[
            pl.BlockSpec((bm, dh), lambda m: (m, 0)),
            pl.BlockSpec((bm, dh), lambda m: (m, 0)),
            pl.BlockSpec((bm, dp), lambda m: (m, 0)),
            pl.BlockSpec((bm, 1), lambda m: (m, 0)),
            pl.BlockSpec((1, dp), lambda m: (0, 0)),
            pl.BlockSpec((dp, dp), lambda m: (0, 0)),
        ],
        out_specs=[
            pl.BlockSpec((bm, dp), lambda m: (m, 0)),
            pl.BlockSpec((bm, dh), lambda m: (m, 0)),
            pl.BlockSpec((bm, dh), lambda m: (m, 0)),
        ],
        out_shape=[
            jax.ShapeDtypeStruct((n, dp), f32),
            jax.ShapeDtypeStruct((n, dh), f32),
            jax.ShapeDtypeStruct((n, dh), f32),
        ],
    )

    s0, s1 = conv_call(y0, y1, epack_c)
    z2, y0, y1 = mid_call(s0, s1, z1, dinv, bp[0], Wp[1])
    s0, s1 = conv_call(y0, y1, epack_c)
    z3, y0, y1 = mid_call(s0, s1, z2, dinv, bp[1], Wp[2])
    s0, s1 = conv_call(y0, y1, epack_c)

    out = pl.pallas_call(
        _fin_body,
        grid=(gm,),
        in_specs=[
            pl.BlockSpec((bm, dh), lambda m: (m, 0)),
            pl.BlockSpec((bm, dh), lambda m: (m, 0)),
            pl.BlockSpec((bm, dp), lambda m: (m, 0)),
            pl.BlockSpec((bm, 1), lambda m: (m, 0)),
            pl.BlockSpec((1, dp), lambda m: (0, 0)),
            pl.BlockSpec((dp, 1), lambda m: (0, 0)),
            pl.BlockSpec((1, 1), lambda m: (0, 0)),
        ],
        out_specs=pl.BlockSpec((bm, 1), lambda m: (m, 0)),
        out_shape=jax.ShapeDtypeStruct((n, 1), f32),
    )(s0, s1, z3, dinv, bp[2], W_out_p, b_out_p)

    return out
